# Initial kernel scaffold; baseline (speedup 1.0000x reference)
import jax
import jax.numpy as jnp
from jax.experimental import pallas as pl


def kernel(category_ids, subcategory_ids, cat_table, sub_table, W_proj, b_proj, W_comb, b_comb):
    raise NotImplementedError("write your pallas kernel here")


# speedup vs baseline: 5.6168x; 5.6168x over previous
import functools

import jax
import jax.numpy as jnp
from jax import lax
from jax.experimental import pallas as pl
from jax.experimental.pallas import tpu as pltpu
from jax.experimental.pallas import tpu_sc as plsc

_NC = 2
_NS = 16
_NW = _NC * _NS
_L = 16

_D = 64
_BLK = 512
_GS = 128


def _sub_fold_body(sub_ref, wpT_ref, wc2T_ref, out_ref):
    tmp = jnp.dot(sub_ref[...], wpT_ref[...], preferred_element_type=jnp.float32)
    out_ref[...] = jnp.dot(tmp, wc2T_ref[...], preferred_element_type=jnp.float32)


def _cat_fold_body(cat_ref, wc1T_ref, wc2T_ref, bp_ref, bc_ref, out_ref):
    bias = bc_ref[...] + jnp.dot(bp_ref[...], wc2T_ref[...],
                                 preferred_element_type=jnp.float32)
    out_ref[...] = jnp.dot(cat_ref[...], wc1T_ref[...],
                           preferred_element_type=jnp.float32) + bias


def _fold_tables(cat_table, sub_table, W_proj, b_proj, W_comb, b_comb):
    wc1T = W_comb[:, :_D].T
    wc2T = W_comb[:, _D:].T
    wpT = W_proj.T
    n_sub, sub_dim = sub_table.shape
    rb = 2000
    sub_contrib = pl.pallas_call(
        _sub_fold_body,
        grid=(n_sub // rb,),
        in_specs=[
            pl.BlockSpec((rb, sub_dim), lambda i: (i, 0)),
            pl.BlockSpec(wpT.shape, lambda i: (0, 0)),
            pl.BlockSpec(wc2T.shape, lambda i: (0, 0)),
        ],
        out_specs=pl.BlockSpec((rb, _D), lambda i: (i, 0)),
        out_shape=jax.ShapeDtypeStruct((n_sub, _D), jnp.float32),
    )(sub_table, wpT, wc2T)
    cat_contrib = pl.pallas_call(
        _cat_fold_body,
        out_shape=jax.ShapeDtypeStruct((cat_table.shape[0], _D), jnp.float32),
    )(cat_table, wc1T, wc2T, b_proj.reshape(1, _D), b_comb.reshape(1, _D))
    return cat_contrib, sub_contrib


@functools.cache
def _make_sc_lookup(n_tokens):
    assert n_tokens % (_NW * _BLK) == 0
    per_w = n_tokens // _NW
    n_blk = per_w // _BLK
    mesh = plsc.VectorSubcoreMesh(core_axis_name="c", subcore_axis_name="s")

    @functools.partial(
        pl.kernel,
        out_type=jax.ShapeDtypeStruct((n_tokens, _D), jnp.float32),
        mesh=mesh,
        scratch_types=[
            pltpu.VMEM((_BLK,), jnp.int32),
            pltpu.VMEM((_BLK,), jnp.int32),
            pltpu.VMEM((_BLK, _D), jnp.float32),
            pltpu.VMEM((_BLK, _D), jnp.float32),
            pltpu.SemaphoreType.DMA,
            pltpu.SemaphoreType.DMA,
        ],
        compiler_params=pltpu.CompilerParams(use_tc_tiling_on_sc=False),
    )
    def sc_lookup(cat_hbm, sub_hbm, cid_hbm, sid_hbm, out_hbm,
                  cidx_v, sidx_v, crow_v, srow_v, gsem, ssem):
        wid = lax.axis_index("s") * _NC + lax.axis_index("c")
        w_base = wid * per_w

        def block_body(b, carry):
            base = w_base + b * _BLK
            pltpu.sync_copy(cid_hbm.at[pl.ds(base, _BLK)], cidx_v)
            pltpu.sync_copy(sid_hbm.at[pl.ds(base, _BLK)], sidx_v)
            copies = []
            for g in range(_BLK // _GS):
                sl = pl.ds(g * _GS, _GS)
                copies.append(
                    pltpu.async_copy(cat_hbm.at[cidx_v.at[sl]], crow_v.at[sl], gsem))
                copies.append(
                    pltpu.async_copy(sub_hbm.at[sidx_v.at[sl]], srow_v.at[sl], ssem))
            for c in copies:
                c.wait()

            def add_body(t, acc):
                for k in range(_D // _L):
                    s2 = pl.ds(k * _L, _L)
                    crow_v[t, s2] = crow_v[t, s2] + srow_v[t, s2]
                return acc

            lax.fori_loop(0, _BLK, add_body, 0)
            pltpu.sync_copy(crow_v, out_hbm.at[pl.ds(base, _BLK)])
            return carry

        lax.fori_loop(0, n_blk, block_body, 0)

    return sc_lookup


def kernel(category_ids, subcategory_ids, cat_table, sub_table,
           W_proj, b_proj, W_comb, b_comb):
    cat_contrib, sub_contrib = _fold_tables(
        cat_table, sub_table, W_proj, b_proj, W_comb, b_comb)
    cid = category_ids.reshape(-1).astype(jnp.int32)
    sid = subcategory_ids.reshape(-1).astype(jnp.int32)
    out = _make_sc_lookup(cid.shape[0])(cat_contrib, sub_contrib, cid, sid)
    return out.reshape(*category_ids.shape, _D)


# speedup vs baseline: 6.0270x; 1.0730x over previous
import functools

import jax
import jax.numpy as jnp
from jax import lax
from jax.experimental import pallas as pl
from jax.experimental.pallas import tpu as pltpu
from jax.experimental.pallas import tpu_sc as plsc

_NC = 2
_NS = 16
_NW = _NC * _NS
_L = 16

_D = 64
_BLK = 512
_GS = 128


def _sub_fold_body(sub_ref, wpT_ref, wc2T_ref, out_ref):
    tmp = jnp.dot(sub_ref[...], wpT_ref[...], preferred_element_type=jnp.float32)
    out_ref[...] = jnp.dot(tmp, wc2T_ref[...], preferred_element_type=jnp.float32)


def _cat_fold_body(cat_ref, wc1T_ref, wc2T_ref, bp_ref, bc_ref, out_ref):
    bias = bc_ref[...] + jnp.dot(bp_ref[...], wc2T_ref[...],
                                 preferred_element_type=jnp.float32)
    out_ref[...] = jnp.dot(cat_ref[...], wc1T_ref[...],
                           preferred_element_type=jnp.float32) + bias


def _fold_tables(cat_table, sub_table, W_proj, b_proj, W_comb, b_comb):
    wc1T = W_comb[:, :_D].T
    wc2T = W_comb[:, _D:].T
    wpT = W_proj.T
    n_sub, sub_dim = sub_table.shape
    rb = 2000
    sub_contrib = pl.pallas_call(
        _sub_fold_body,
        grid=(n_sub // rb,),
        in_specs=[
            pl.BlockSpec((rb, sub_dim), lambda i: (i, 0)),
            pl.BlockSpec(wpT.shape, lambda i: (0, 0)),
            pl.BlockSpec(wc2T.shape, lambda i: (0, 0)),
        ],
        out_specs=pl.BlockSpec((rb, _D), lambda i: (i, 0)),
        out_shape=jax.ShapeDtypeStruct((n_sub, _D), jnp.float32),
    )(sub_table, wpT, wc2T)
    cat_contrib = pl.pallas_call(
        _cat_fold_body,
        out_shape=jax.ShapeDtypeStruct((cat_table.shape[0], _D), jnp.float32),
    )(cat_table, wc1T, wc2T, b_proj.reshape(1, _D), b_comb.reshape(1, _D))
    return cat_contrib, sub_contrib


@functools.cache
def _make_sc_lookup(n_tokens):
    blk = 256
    ng = blk // _GS
    assert n_tokens % (_NW * 2 * blk) == 0
    per_w = n_tokens // _NW
    n_blk = per_w // blk
    mesh = plsc.VectorSubcoreMesh(core_axis_name="c", subcore_axis_name="s")

    @functools.partial(
        pl.kernel,
        out_type=jax.ShapeDtypeStruct((n_tokens, _D), jnp.float32),
        mesh=mesh,
        scratch_types=[
            pltpu.VMEM((blk,), jnp.int32), pltpu.VMEM((blk,), jnp.int32),
            pltpu.VMEM((blk,), jnp.int32), pltpu.VMEM((blk,), jnp.int32),
            pltpu.VMEM((blk, _D), jnp.float32), pltpu.VMEM((blk, _D), jnp.float32),
            pltpu.VMEM((blk, _D), jnp.float32), pltpu.VMEM((blk, _D), jnp.float32),
            pltpu.SemaphoreType.DMA, pltpu.SemaphoreType.DMA,
            pltpu.SemaphoreType.DMA, pltpu.SemaphoreType.DMA,
            pltpu.SemaphoreType.DMA, pltpu.SemaphoreType.DMA,
        ],
        compiler_params=pltpu.CompilerParams(use_tc_tiling_on_sc=False),
    )
    def sc_lookup(cat_hbm, sub_hbm, cid_hbm, sid_hbm, out_hbm,
                  cidx0, sidx0, cidx1, sidx1, crow0, srow0, crow1, srow1,
                  gsem0, gsem1, isem0, isem1, osem0, osem1):
        wid = lax.axis_index("s") * _NC + lax.axis_index("c")
        w_base = wid * per_w
        bufs = ((cidx0, sidx0, crow0, srow0, gsem0, isem0, osem0),
                (cidx1, sidx1, crow1, srow1, gsem1, isem1, osem1))

        def issue_idx(i, p):
            cidx, sidx, _, _, _, isem, _ = bufs[p]
            base = w_base + i * blk
            pltpu.async_copy(cid_hbm.at[pl.ds(base, blk)], cidx, isem)
            pltpu.async_copy(sid_hbm.at[pl.ds(base, blk)], sidx, isem)

        def wait_idx(i, p):
            cidx, sidx, _, _, _, isem, _ = bufs[p]
            base = w_base + i * blk
            pltpu.make_async_copy(cid_hbm.at[pl.ds(base, blk)], cidx, isem).wait()
            pltpu.make_async_copy(sid_hbm.at[pl.ds(base, blk)], sidx, isem).wait()

        def issue_gathers(p):
            cidx, sidx, crow, srow, gsem, _, _ = bufs[p]
            for g in range(ng):
                sl = pl.ds(g * _GS, _GS)
                pltpu.async_copy(cat_hbm.at[cidx.at[sl]], crow.at[sl], gsem)
                pltpu.async_copy(sub_hbm.at[sidx.at[sl]], srow.at[sl], gsem)

        def wait_gathers(p):
            cidx, sidx, crow, srow, gsem, _, _ = bufs[p]
            for g in range(ng):
                sl = pl.ds(g * _GS, _GS)
                pltpu.make_async_copy(cat_hbm.at[cidx.at[sl]], crow.at[sl], gsem).wait()
                pltpu.make_async_copy(sub_hbm.at[sidx.at[sl]], srow.at[sl], gsem).wait()

        def issue_store(i, p):
            _, _, crow, _, _, _, osem = bufs[p]
            base = w_base + i * blk
            pltpu.async_copy(crow, out_hbm.at[pl.ds(base, blk)], osem)

        def wait_store(i, p):
            _, _, crow, _, _, _, osem = bufs[p]
            base = w_base + i * blk
            pltpu.make_async_copy(crow, out_hbm.at[pl.ds(base, blk)], osem).wait()

        def do_add(p):
            _, _, crow, srow, _, _, _ = bufs[p]

            @plsc.parallel_loop(0, blk, 4)
            def _(t):
                for dt in range(4):
                    tt = t + dt
                    for k in range(_D // _L):
                        s2 = pl.ds(k * _L, _L)
                        crow[tt, s2] = crow[tt, s2] + srow[tt, s2]

        base0 = w_base
        pltpu.sync_copy(cid_hbm.at[pl.ds(base0, blk)], cidx0)
        pltpu.sync_copy(sid_hbm.at[pl.ds(base0, blk)], sidx0)
        issue_gathers(0)
        pltpu.sync_copy(cid_hbm.at[pl.ds(base0 + blk, blk)], cidx1)
        pltpu.sync_copy(sid_hbm.at[pl.ds(base0 + blk, blk)], sidx1)

        @pl.loop(0, n_blk // 2)
        def _(o):
            for p in (0, 1):
                i = 2 * o + p
                q = 1 - p

                @pl.when(i + 1 < n_blk)
                def _():
                    @pl.when(i >= 1)
                    def _():
                        wait_store(i - 1, q)
                        wait_idx(i + 1, q)
                    issue_gathers(q)

                wait_gathers(p)

                @pl.when(i + 2 < n_blk)
                def _():
                    issue_idx(i + 2, p)

                do_add(p)
                issue_store(i, p)

        wait_store(n_blk - 2, (n_blk - 2) % 2)
        wait_store(n_blk - 1, (n_blk - 1) % 2)

    return sc_lookup


def kernel(category_ids, subcategory_ids, cat_table, sub_table,
           W_proj, b_proj, W_comb, b_comb):
    cat_contrib, sub_contrib = _fold_tables(
        cat_table, sub_table, W_proj, b_proj, W_comb, b_comb)
    cid = category_ids.reshape(-1).astype(jnp.int32)
    sid = subcategory_ids.reshape(-1).astype(jnp.int32)
    out = _make_sc_lookup(cid.shape[0])(cat_contrib, sub_contrib, cid, sid)
    return out.reshape(*category_ids.shape, _D)


# speedup vs baseline: 8.6011x; 1.4271x over previous
import functools

import jax
import jax.numpy as jnp
from jax import lax
from jax.experimental import pallas as pl
from jax.experimental.pallas import tpu as pltpu
from jax.experimental.pallas import tpu_sc as plsc

_NC = 2
_NS = 16
_NW = _NC * _NS
_L = 16

_D = 64
_BLK = 512
_GS = 128


def _sub_fold_body(sub_ref, wpT_ref, wc2T_ref, out_ref):
    tmp = jnp.dot(sub_ref[...], wpT_ref[...], preferred_element_type=jnp.float32)
    out_ref[...] = jnp.dot(tmp, wc2T_ref[...], preferred_element_type=jnp.float32)


def _cat_fold_body(cat_ref, wc1T_ref, wc2T_ref, bp_ref, bc_ref, out_ref):
    bias = bc_ref[...] + jnp.dot(bp_ref[...], wc2T_ref[...],
                                 preferred_element_type=jnp.float32)
    out_ref[...] = jnp.dot(cat_ref[...], wc1T_ref[...],
                           preferred_element_type=jnp.float32) + bias


def _fold_tables(cat_table, sub_table, W_proj, b_proj, W_comb, b_comb):
    wc1T = W_comb[:, :_D].T
    wc2T = W_comb[:, _D:].T
    wpT = W_proj.T
    n_sub, sub_dim = sub_table.shape
    rb = 2000
    sub_contrib = pl.pallas_call(
        _sub_fold_body,
        grid=(n_sub // rb,),
        in_specs=[
            pl.BlockSpec((rb, sub_dim), lambda i: (i, 0)),
            pl.BlockSpec(wpT.shape, lambda i: (0, 0)),
            pl.BlockSpec(wc2T.shape, lambda i: (0, 0)),
        ],
        out_specs=pl.BlockSpec((rb, _D), lambda i: (i, 0)),
        out_shape=jax.ShapeDtypeStruct((n_sub, _D), jnp.float32),
    )(sub_table, wpT, wc2T)
    cat_contrib = pl.pallas_call(
        _cat_fold_body,
        out_shape=jax.ShapeDtypeStruct((cat_table.shape[0], _D), jnp.float32),
    )(cat_table, wc1T, wc2T, b_proj.reshape(1, _D), b_comb.reshape(1, _D))
    return cat_contrib, sub_contrib


@functools.cache
def _make_sc_lookup(n_tokens):
    blk = 256
    ng = blk // _GS
    assert n_tokens % (_NW * 2 * blk) == 0
    per_w = n_tokens // _NW
    n_blk = per_w // blk
    mesh = plsc.VectorSubcoreMesh(core_axis_name="c", subcore_axis_name="s")

    @functools.partial(
        pl.kernel,
        out_type=jax.ShapeDtypeStruct((n_tokens, 2 * _D), jnp.float32),
        mesh=mesh,
        scratch_types=[
            pltpu.VMEM((blk,), jnp.int32), pltpu.VMEM((blk,), jnp.int32),
            pltpu.VMEM((blk,), jnp.int32), pltpu.VMEM((blk,), jnp.int32),
            pltpu.VMEM((blk, _D), jnp.float32), pltpu.VMEM((blk, _D), jnp.float32),
            pltpu.VMEM((blk, _D), jnp.float32), pltpu.VMEM((blk, _D), jnp.float32),
            pltpu.SemaphoreType.DMA, pltpu.SemaphoreType.DMA,
            pltpu.SemaphoreType.DMA, pltpu.SemaphoreType.DMA,
            pltpu.SemaphoreType.DMA, pltpu.SemaphoreType.DMA,
        ],
        compiler_params=pltpu.CompilerParams(use_tc_tiling_on_sc=False),
    )
    def sc_lookup(cat_hbm, sub_hbm, cid_hbm, sid_hbm, out_hbm,
                  cidx0, sidx0, cidx1, sidx1, crow0, srow0, crow1, srow1,
                  gsem0, gsem1, isem0, isem1, osem0, osem1):
        wid = lax.axis_index("s") * _NC + lax.axis_index("c")
        w_base = wid * per_w
        bufs = ((cidx0, sidx0, crow0, srow0, gsem0, isem0, osem0),
                (cidx1, sidx1, crow1, srow1, gsem1, isem1, osem1))

        def issue_idx(i, p):
            cidx, sidx, _, _, _, isem, _ = bufs[p]
            base = w_base + i * blk
            pltpu.async_copy(cid_hbm.at[pl.ds(base, blk)], cidx, isem)
            pltpu.async_copy(sid_hbm.at[pl.ds(base, blk)], sidx, isem)

        def wait_idx(i, p):
            cidx, sidx, _, _, _, isem, _ = bufs[p]
            base = w_base + i * blk
            pltpu.make_async_copy(cid_hbm.at[pl.ds(base, blk)], cidx, isem).wait()
            pltpu.make_async_copy(sid_hbm.at[pl.ds(base, blk)], sidx, isem).wait()

        def issue_gathers(p):
            cidx, sidx, crow, srow, gsem, _, _ = bufs[p]
            for g in range(ng):
                sl = pl.ds(g * _GS, _GS)
                pltpu.async_copy(cat_hbm.at[cidx.at[sl]], crow.at[sl], gsem)
                pltpu.async_copy(sub_hbm.at[sidx.at[sl]], srow.at[sl], gsem)

        def wait_gathers(p):
            cidx, sidx, crow, srow, gsem, _, _ = bufs[p]
            for g in range(ng):
                sl = pl.ds(g * _GS, _GS)
                pltpu.make_async_copy(cat_hbm.at[cidx.at[sl]], crow.at[sl], gsem).wait()
                pltpu.make_async_copy(sub_hbm.at[sidx.at[sl]], srow.at[sl], gsem).wait()

        def issue_store(i, p):
            _, _, crow, _, _, _, osem = bufs[p]
            base = w_base + i * blk
            pltpu.async_copy(crow, out_hbm.at[pl.ds(base, blk), pl.ds(0, _D)], osem)

        def wait_store(i, p):
            _, _, crow, _, _, _, osem = bufs[p]
            base = w_base + i * blk
            pltpu.make_async_copy(
                crow, out_hbm.at[pl.ds(base, blk), pl.ds(0, _D)], osem).wait()

        def do_add(p):
            _, _, crow, srow, _, _, _ = bufs[p]

            @plsc.parallel_loop(0, blk, 4)
            def _(t):
                for dt in range(4):
                    tt = t + dt
                    for k in range(_D // _L):
                        s2 = pl.ds(k * _L, _L)
                        crow[tt, s2] = crow[tt, s2] + srow[tt, s2]

        base0 = w_base
        pltpu.sync_copy(cid_hbm.at[pl.ds(base0, blk)], cidx0)
        pltpu.sync_copy(sid_hbm.at[pl.ds(base0, blk)], sidx0)
        issue_gathers(0)
        pltpu.sync_copy(cid_hbm.at[pl.ds(base0 + blk, blk)], cidx1)
        pltpu.sync_copy(sid_hbm.at[pl.ds(base0 + blk, blk)], sidx1)

        @pl.loop(0, n_blk // 2)
        def _(o):
            for p in (0, 1):
                i = 2 * o + p
                q = 1 - p

                @pl.when(i + 1 < n_blk)
                def _():
                    @pl.when(i >= 1)
                    def _():
                        wait_store(i - 1, q)
                        wait_idx(i + 1, q)
                    issue_gathers(q)

                wait_gathers(p)

                @pl.when(i + 2 < n_blk)
                def _():
                    issue_idx(i + 2, p)

                do_add(p)
                issue_store(i, p)

        wait_store(n_blk - 2, (n_blk - 2) % 2)
        wait_store(n_blk - 1, (n_blk - 1) % 2)

    return sc_lookup


def kernel(category_ids, subcategory_ids, cat_table, sub_table,
           W_proj, b_proj, W_comb, b_comb):
    cat_contrib, sub_contrib = _fold_tables(
        cat_table, sub_table, W_proj, b_proj, W_comb, b_comb)
    cid = category_ids.reshape(-1).astype(jnp.int32)
    sid = subcategory_ids.reshape(-1).astype(jnp.int32)
    out = _make_sc_lookup(cid.shape[0])(cat_contrib, sub_contrib, cid, sid)
    b, s = category_ids.shape
    return out.reshape(b, s, 2 * _D)[..., :_D]
